# baseline (speedup 1.0000x reference)
import jax
import jax.numpy as jnp
from jax import lax
from jax.experimental import pallas as pl
from jax.experimental.pallas import tpu as pltpu
from jax.experimental.pallas import tpu_sc as plsc

E = 8
TOPK = 2
H = 768
I = 3072
T = 2048

BM = 128
P = TOPK * T + E * BM
NB = P // BM
BI = 1024
NI = I // BI
BT = 256

NC = 2
NS = 16
NW = NC * NS
SC_CHUNK = 32


def _k0_body(hs_hbm, tok_hbm, xs_out, idx_v, rows_v, sem):
    wid = lax.axis_index("s") * NC + lax.axis_index("c")
    rows_per_w = P // NW
    for ch in range(rows_per_w // SC_CHUNK):
        base = wid * rows_per_w + ch * SC_CHUNK
        pltpu.sync_copy(tok_hbm.at[pl.ds(base, SC_CHUNK)], idx_v)
        pltpu.async_copy(hs_hbm.at[idx_v], rows_v, sem).wait()
        pltpu.sync_copy(rows_v, xs_out.at[pl.ds(base, SC_CHUNK)])


def _k3_body(y_hbm, pos0_hbm, pos1_hbm, y0_out, y1_out,
             p0_v, p1_v, r0_v, r1_v, sem):
    wid = lax.axis_index("s") * NC + lax.axis_index("c")
    toks_per_w = T // NW
    for ch in range(toks_per_w // SC_CHUNK):
        base = wid * toks_per_w + ch * SC_CHUNK
        pltpu.sync_copy(pos0_hbm.at[pl.ds(base, SC_CHUNK)], p0_v)
        pltpu.sync_copy(pos1_hbm.at[pl.ds(base, SC_CHUNK)], p1_v)
        c0 = pltpu.async_copy(y_hbm.at[p0_v], r0_v, sem)
        c1 = pltpu.async_copy(y_hbm.at[p1_v], r1_v, sem)
        c0.wait()
        c1.wait()
        pltpu.sync_copy(r0_v, y0_out.at[pl.ds(base, SC_CHUNK)])
        pltpu.sync_copy(r1_v, y1_out.at[pl.ds(base, SC_CHUNK)])


def _k1_body(be_ref, x_ref, wg_ref, wu_ref, o_ref):
    g = jnp.dot(x_ref[...], wg_ref[0], preferred_element_type=jnp.float32)
    u = jnp.dot(x_ref[...], wu_ref[0], preferred_element_type=jnp.float32)
    o_ref[...] = g * lax.logistic(g) * u


def _k2_body(be_ref, inter_ref, wd_ref, o_ref):
    o_ref[...] = jnp.dot(inter_ref[...], wd_ref[0],
                         preferred_element_type=jnp.float32)


def _k4_body(y0_ref, y1_ref, aff_ref, idx_ref, o_ref):
    aff = aff_ref[...]
    i0 = idx_ref[:, 0:1]
    i1 = idx_ref[:, 1:2]
    a0 = jnp.zeros((BT, 1), jnp.float32)
    a1 = jnp.zeros((BT, 1), jnp.float32)
    for e in range(E):
        a0 = a0 + jnp.where(i0 == e, aff[:, e:e + 1], 0.0)
        a1 = a1 + jnp.where(i1 == e, aff[:, e:e + 1], 0.0)
    dup = i0 == i1
    denom = jnp.abs(a0) + jnp.where(dup, 0.0, jnp.abs(a1))
    denom = jnp.maximum(denom, 1e-12)
    w0 = a0 / denom
    w1 = jnp.where(dup, 0.0, a1 / denom)
    o_ref[...] = w0 * y0_ref[...] + w1 * y1_ref[...]


def _sc_gather_rows(hidden_states, tok_map):
    mesh = plsc.VectorSubcoreMesh(core_axis_name="c", subcore_axis_name="s")
    return pl.kernel(
        _k0_body,
        mesh=mesh,
        out_type=jax.ShapeDtypeStruct((P, H), jnp.float32),
        scratch_types=[
            pltpu.VMEM((SC_CHUNK,), jnp.int32),
            pltpu.VMEM((SC_CHUNK, H), jnp.float32),
            pltpu.SemaphoreType.DMA,
        ],
    )(hidden_states, tok_map)


def _sc_gather_pair(y, pos0, pos1):
    mesh = plsc.VectorSubcoreMesh(core_axis_name="c", subcore_axis_name="s")
    return pl.kernel(
        _k3_body,
        mesh=mesh,
        out_type=[
            jax.ShapeDtypeStruct((T, H), jnp.float32),
            jax.ShapeDtypeStruct((T, H), jnp.float32),
        ],
        scratch_types=[
            pltpu.VMEM((SC_CHUNK,), jnp.int32),
            pltpu.VMEM((SC_CHUNK,), jnp.int32),
            pltpu.VMEM((SC_CHUNK, H), jnp.float32),
            pltpu.VMEM((SC_CHUNK, H), jnp.float32),
            pltpu.SemaphoreType.DMA,
        ],
    )(y, pos0, pos1)


def _tc_gate_up(block_expert, x_sorted, W_gate_up, interpret=False):
    grid_spec = pltpu.PrefetchScalarGridSpec(
        num_scalar_prefetch=1,
        grid=(NI, NB),
        in_specs=[
            pl.BlockSpec((BM, H), lambda j, b, be: (b, 0)),
            pl.BlockSpec((1, H, BI), lambda j, b, be: (be[b], 0, j)),
            pl.BlockSpec((1, H, BI), lambda j, b, be: (be[b], 0, NI + j)),
        ],
        out_specs=pl.BlockSpec((BM, BI), lambda j, b, be: (b, j)),
    )
    return pl.pallas_call(
        _k1_body,
        grid_spec=grid_spec,
        out_shape=jax.ShapeDtypeStruct((P, I), jnp.float32),
        compiler_params=pltpu.CompilerParams(
            dimension_semantics=("arbitrary", "arbitrary")),
        interpret=interpret,
    )(block_expert, x_sorted, W_gate_up, W_gate_up)


def _tc_down(block_expert, inter, W_down, interpret=False):
    grid_spec = pltpu.PrefetchScalarGridSpec(
        num_scalar_prefetch=1,
        grid=(NB,),
        in_specs=[
            pl.BlockSpec((BM, I), lambda b, be: (b, 0)),
            pl.BlockSpec((1, I, H), lambda b, be: (be[b], 0, 0)),
        ],
        out_specs=pl.BlockSpec((BM, H), lambda b, be: (b, 0)),
    )
    return pl.pallas_call(
        _k2_body,
        grid_spec=grid_spec,
        out_shape=jax.ShapeDtypeStruct((P, H), jnp.float32),
        compiler_params=pltpu.CompilerParams(
            dimension_semantics=("arbitrary",)),
        interpret=interpret,
    )(block_expert, inter, W_down)


def _tc_combine(y0, y1, expert_affinities, idx32, interpret=False):
    return pl.pallas_call(
        _k4_body,
        grid=(T // BT,),
        in_specs=[
            pl.BlockSpec((BT, H), lambda b: (b, 0)),
            pl.BlockSpec((BT, H), lambda b: (b, 0)),
            pl.BlockSpec((BT, E), lambda b: (b, 0)),
            pl.BlockSpec((BT, TOPK), lambda b: (b, 0)),
        ],
        out_specs=pl.BlockSpec((BT, H), lambda b: (b, 0)),
        out_shape=jax.ShapeDtypeStruct((T, H), jnp.float32),
        interpret=interpret,
    )(y0, y1, expert_affinities, idx32)


def _routing_metadata(idx32):
    flat_e = idx32.reshape(-1)
    n = flat_e.shape[0]
    perm = jnp.argsort(flat_e).astype(jnp.int32)
    sorted_e = flat_e[perm]
    counts = jnp.bincount(flat_e, length=E).astype(jnp.int32)
    padded = ((counts + BM - 1) // BM) * BM
    pend = jnp.cumsum(padded)
    pstart = pend - padded
    gend = jnp.cumsum(counts)
    gstart = gend - counts
    dest = (pstart[sorted_e] + jnp.arange(n, dtype=jnp.int32)
            - gstart[sorted_e]).astype(jnp.int32)
    tok_map = jnp.zeros((P,), jnp.int32).at[dest].set(perm // 2)
    pos_flat = jnp.zeros((n,), jnp.int32).at[perm].set(dest)
    pos0 = pos_flat[0::2]
    pos1 = pos_flat[1::2]
    block_starts = jnp.arange(NB, dtype=jnp.int32) * BM
    block_expert = jnp.minimum(
        jnp.searchsorted(pend, block_starts, side="right"), E - 1
    ).astype(jnp.int32)
    return tok_map, pos0, pos1, block_expert


def kernel(hidden_states, expert_affinities, expert_index, W_gate_up, W_down):
    idx32 = expert_index.astype(jnp.int32)
    tok_map, pos0, pos1, block_expert = _routing_metadata(idx32)
    x_sorted = _sc_gather_rows(hidden_states, tok_map)
    inter = _tc_gate_up(block_expert, x_sorted, W_gate_up)
    y = _tc_down(block_expert, inter, W_down)
    y0, y1 = _sc_gather_pair(y, pos0, pos1)
    return _tc_combine(y0, y1, expert_affinities, idx32)


# speedup vs baseline: 1.0258x; 1.0258x over previous
import jax
import jax.numpy as jnp
from jax import lax
from jax.experimental import pallas as pl
from jax.experimental.pallas import tpu as pltpu
from jax.experimental.pallas import tpu_sc as plsc

E = 8
TOPK = 2
H = 768
I = 3072
T = 2048

BM = 128
P = TOPK * T + E * BM
NB = P // BM
BI = 1024
NI = I // BI
BT = 256

NC = 2
NS = 16
NW = NC * NS
SC_CHUNK = 32


def _k0_body(hs_hbm, tok_hbm, xs_out, idx_v, rows_v, sem):
    wid = lax.axis_index("s") * NC + lax.axis_index("c")
    rows_per_w = P // NW
    base = wid * rows_per_w
    pltpu.sync_copy(tok_hbm.at[pl.ds(base, rows_per_w)], idx_v)
    c0 = pltpu.async_copy(hs_hbm.at[idx_v.at[pl.ds(0, 128)]],
                          rows_v.at[pl.ds(0, 128)], sem)
    c1 = pltpu.async_copy(hs_hbm.at[idx_v.at[pl.ds(128, 32)]],
                          rows_v.at[pl.ds(128, 32)], sem)
    c0.wait()
    c1.wait()
    pltpu.sync_copy(rows_v, xs_out.at[pl.ds(base, rows_per_w)])


def _k3_body(y_hbm, pos0_hbm, pos1_hbm, y0_out, y1_out,
             p0_v, p1_v, r0_v, r1_v, sem):
    wid = lax.axis_index("s") * NC + lax.axis_index("c")
    toks_per_w = T // NW
    base = wid * toks_per_w
    pltpu.sync_copy(pos0_hbm.at[pl.ds(base, toks_per_w)], p0_v)
    pltpu.sync_copy(pos1_hbm.at[pl.ds(base, toks_per_w)], p1_v)
    c0 = pltpu.async_copy(y_hbm.at[p0_v], r0_v, sem)
    c1 = pltpu.async_copy(y_hbm.at[p1_v], r1_v, sem)
    c0.wait()
    pltpu.sync_copy(r0_v, y0_out.at[pl.ds(base, toks_per_w)])
    c1.wait()
    pltpu.sync_copy(r1_v, y1_out.at[pl.ds(base, toks_per_w)])


def _k1_body(be_ref, x_ref, wg_ref, wu_ref, o_ref):
    x = x_ref[...].astype(jnp.bfloat16)
    g = jnp.dot(x, wg_ref[0].astype(jnp.bfloat16),
                preferred_element_type=jnp.float32)
    u = jnp.dot(x, wu_ref[0].astype(jnp.bfloat16),
                preferred_element_type=jnp.float32)
    o_ref[...] = (g * lax.logistic(g) * u).astype(jnp.bfloat16)


def _k2_body(be_ref, inter_ref, wd_ref, o_ref):
    o_ref[...] = jnp.dot(inter_ref[...], wd_ref[0].astype(jnp.bfloat16),
                         preferred_element_type=jnp.float32)


def _k4_body(y0_ref, y1_ref, aff_ref, idx_ref, o_ref):
    aff = aff_ref[...]
    i0 = idx_ref[:, 0:1]
    i1 = idx_ref[:, 1:2]
    a0 = jnp.zeros((BT, 1), jnp.float32)
    a1 = jnp.zeros((BT, 1), jnp.float32)
    for e in range(E):
        a0 = a0 + jnp.where(i0 == e, aff[:, e:e + 1], 0.0)
        a1 = a1 + jnp.where(i1 == e, aff[:, e:e + 1], 0.0)
    dup = i0 == i1
    denom = jnp.abs(a0) + jnp.where(dup, 0.0, jnp.abs(a1))
    denom = jnp.maximum(denom, 1e-12)
    w0 = a0 / denom
    w1 = jnp.where(dup, 0.0, a1 / denom)
    o_ref[...] = w0 * y0_ref[...] + w1 * y1_ref[...]


def _sc_gather_rows(hidden_states, tok_map):
    mesh = plsc.VectorSubcoreMesh(core_axis_name="c", subcore_axis_name="s")
    return pl.kernel(
        _k0_body,
        mesh=mesh,
        out_type=jax.ShapeDtypeStruct((P, H), jnp.float32),
        scratch_types=[
            pltpu.VMEM((P // NW,), jnp.int32),
            pltpu.VMEM((P // NW, H), jnp.float32),
            pltpu.SemaphoreType.DMA,
        ],
    )(hidden_states, tok_map)


def _sc_gather_pair(y, pos0, pos1):
    mesh = plsc.VectorSubcoreMesh(core_axis_name="c", subcore_axis_name="s")
    return pl.kernel(
        _k3_body,
        mesh=mesh,
        out_type=[
            jax.ShapeDtypeStruct((T, H), jnp.float32),
            jax.ShapeDtypeStruct((T, H), jnp.float32),
        ],
        scratch_types=[
            pltpu.VMEM((T // NW,), jnp.int32),
            pltpu.VMEM((T // NW,), jnp.int32),
            pltpu.VMEM((T // NW, H), jnp.float32),
            pltpu.VMEM((T // NW, H), jnp.float32),
            pltpu.SemaphoreType.DMA,
        ],
    )(y, pos0, pos1)


def _tc_gate_up(block_expert, x_sorted, W_gate_up, interpret=False):
    grid_spec = pltpu.PrefetchScalarGridSpec(
        num_scalar_prefetch=1,
        grid=(NI, NB),
        in_specs=[
            pl.BlockSpec((BM, H), lambda j, b, be: (b, 0)),
            pl.BlockSpec((1, H, BI), lambda j, b, be: (be[b], 0, j)),
            pl.BlockSpec((1, H, BI), lambda j, b, be: (be[b], 0, NI + j)),
        ],
        out_specs=pl.BlockSpec((BM, BI), lambda j, b, be: (b, j)),
    )
    return pl.pallas_call(
        _k1_body,
        grid_spec=grid_spec,
        out_shape=jax.ShapeDtypeStruct((P, I), jnp.bfloat16),
        compiler_params=pltpu.CompilerParams(
            dimension_semantics=("arbitrary", "arbitrary")),
        interpret=interpret,
    )(block_expert, x_sorted, W_gate_up, W_gate_up)


def _tc_down(block_expert, inter, W_down, interpret=False):
    grid_spec = pltpu.PrefetchScalarGridSpec(
        num_scalar_prefetch=1,
        grid=(NB,),
        in_specs=[
            pl.BlockSpec((BM, I), lambda b, be: (b, 0)),
            pl.BlockSpec((1, I, H), lambda b, be: (be[b], 0, 0)),
        ],
        out_specs=pl.BlockSpec((BM, H), lambda b, be: (b, 0)),
    )
    return pl.pallas_call(
        _k2_body,
        grid_spec=grid_spec,
        out_shape=jax.ShapeDtypeStruct((P, H), jnp.float32),
        compiler_params=pltpu.CompilerParams(
            dimension_semantics=("arbitrary",)),
        interpret=interpret,
    )(block_expert, inter, W_down)


def _tc_combine(y0, y1, expert_affinities, idx32, interpret=False):
    return pl.pallas_call(
        _k4_body,
        grid=(T // BT,),
        in_specs=[
            pl.BlockSpec((BT, H), lambda b: (b, 0)),
            pl.BlockSpec((BT, H), lambda b: (b, 0)),
            pl.BlockSpec((BT, E), lambda b: (b, 0)),
            pl.BlockSpec((BT, TOPK), lambda b: (b, 0)),
        ],
        out_specs=pl.BlockSpec((BT, H), lambda b: (b, 0)),
        out_shape=jax.ShapeDtypeStruct((T, H), jnp.float32),
        interpret=interpret,
    )(y0, y1, expert_affinities, idx32)


def _routing_metadata(idx32):
    flat_e = idx32.reshape(-1)
    n = flat_e.shape[0]
    perm = jnp.argsort(flat_e).astype(jnp.int32)
    sorted_e = flat_e[perm]
    counts = jnp.bincount(flat_e, length=E).astype(jnp.int32)
    padded = ((counts + BM - 1) // BM) * BM
    pend = jnp.cumsum(padded)
    pstart = pend - padded
    gend = jnp.cumsum(counts)
    gstart = gend - counts
    dest = (pstart[sorted_e] + jnp.arange(n, dtype=jnp.int32)
            - gstart[sorted_e]).astype(jnp.int32)
    tok_map = jnp.zeros((P,), jnp.int32).at[dest].set(perm // 2)
    pos_flat = jnp.zeros((n,), jnp.int32).at[perm].set(dest)
    pos0 = pos_flat[0::2]
    pos1 = pos_flat[1::2]
    block_starts = jnp.arange(NB, dtype=jnp.int32) * BM
    block_expert = jnp.minimum(
        jnp.searchsorted(pend, block_starts, side="right"), E - 1
    ).astype(jnp.int32)
    return tok_map, pos0, pos1, block_expert


def kernel(hidden_states, expert_affinities, expert_index, W_gate_up, W_down):
    idx32 = expert_index.astype(jnp.int32)
    tok_map, pos0, pos1, block_expert = _routing_metadata(idx32)
    x_sorted = _sc_gather_rows(hidden_states, tok_map)
    inter = _tc_gate_up(block_expert, x_sorted, W_gate_up)
    y = _tc_down(block_expert, inter, W_down)
    y0, y1 = _sc_gather_pair(y, pos0, pos1)
    return _tc_combine(y0, y1, expert_affinities, idx32)


# speedup vs baseline: 1.1054x; 1.0776x over previous
import jax
import jax.numpy as jnp
from jax import lax
from jax.experimental import pallas as pl
from jax.experimental.pallas import tpu as pltpu
from jax.experimental.pallas import tpu_sc as plsc

E = 8
TOPK = 2
H = 768
I = 3072
T = 2048

BM = 128
P = TOPK * T + E * BM
NB = P // BM
BI = 1024
NI = I // BI
BT = 256

NC = 2
NS = 16
NW = NC * NS
SC_CHUNK = 32


def _k0_body(hs_hbm, tok_hbm, xs_out, idx_v, rows_v, sem):
    wid = lax.axis_index("s") * NC + lax.axis_index("c")
    rows_per_w = P // NW
    base = wid * rows_per_w
    pltpu.sync_copy(tok_hbm.at[pl.ds(base, rows_per_w)], idx_v)
    c0 = pltpu.async_copy(hs_hbm.at[idx_v.at[pl.ds(0, 128)]],
                          rows_v.at[pl.ds(0, 128)], sem)
    c1 = pltpu.async_copy(hs_hbm.at[idx_v.at[pl.ds(128, 32)]],
                          rows_v.at[pl.ds(128, 32)], sem)
    c0.wait()
    c1.wait()
    pltpu.sync_copy(rows_v, xs_out.at[pl.ds(base, rows_per_w)])


def _k3_body(y_hbm, pos0_hbm, pos1_hbm, y0_out, y1_out,
             p0_v, p1_v, r0_v, r1_v, sem):
    wid = lax.axis_index("s") * NC + lax.axis_index("c")
    toks_per_w = T // NW
    base = wid * toks_per_w
    pltpu.sync_copy(pos0_hbm.at[pl.ds(base, toks_per_w)], p0_v)
    pltpu.sync_copy(pos1_hbm.at[pl.ds(base, toks_per_w)], p1_v)
    c0 = pltpu.async_copy(y_hbm.at[p0_v], r0_v, sem)
    c1 = pltpu.async_copy(y_hbm.at[p1_v], r1_v, sem)
    c0.wait()
    pltpu.sync_copy(r0_v, y0_out.at[pl.ds(base, toks_per_w)])
    c1.wait()
    pltpu.sync_copy(r1_v, y1_out.at[pl.ds(base, toks_per_w)])


def _k1_body(be_ref, x_ref, wg_ref, wu_ref, o_ref):
    x = x_ref[...].astype(jnp.bfloat16)
    g = jnp.dot(x, wg_ref[0].astype(jnp.bfloat16),
                preferred_element_type=jnp.float32)
    u = jnp.dot(x, wu_ref[0].astype(jnp.bfloat16),
                preferred_element_type=jnp.float32)
    o_ref[...] = (g * lax.logistic(g) * u).astype(jnp.bfloat16)


def _k2_body(be_ref, inter_ref, wd_ref, o_ref):
    o_ref[...] = jnp.dot(inter_ref[...], wd_ref[0].astype(jnp.bfloat16),
                         preferred_element_type=jnp.float32)


def _k4_body(y0_ref, y1_ref, aff_ref, idx_ref, o_ref):
    aff = aff_ref[...]
    i0 = idx_ref[:, 0:1]
    i1 = idx_ref[:, 1:2]
    a0 = jnp.zeros((BT, 1), jnp.float32)
    a1 = jnp.zeros((BT, 1), jnp.float32)
    for e in range(E):
        a0 = a0 + jnp.where(i0 == e, aff[:, e:e + 1], 0.0)
        a1 = a1 + jnp.where(i1 == e, aff[:, e:e + 1], 0.0)
    dup = i0 == i1
    denom = jnp.abs(a0) + jnp.where(dup, 0.0, jnp.abs(a1))
    denom = jnp.maximum(denom, 1e-12)
    w0 = a0 / denom
    w1 = jnp.where(dup, 0.0, a1 / denom)
    o_ref[...] = w0 * y0_ref[...] + w1 * y1_ref[...]


def _sc_gather_rows(hidden_states, tok_map):
    mesh = plsc.VectorSubcoreMesh(core_axis_name="c", subcore_axis_name="s")
    return pl.kernel(
        _k0_body,
        mesh=mesh,
        out_type=jax.ShapeDtypeStruct((P, H), jnp.float32),
        scratch_types=[
            pltpu.VMEM((P // NW,), jnp.int32),
            pltpu.VMEM((P // NW, H), jnp.float32),
            pltpu.SemaphoreType.DMA,
        ],
    )(hidden_states, tok_map)


def _sc_gather_pair(y, pos0, pos1):
    mesh = plsc.VectorSubcoreMesh(core_axis_name="c", subcore_axis_name="s")
    return pl.kernel(
        _k3_body,
        mesh=mesh,
        out_type=[
            jax.ShapeDtypeStruct((T, H), jnp.float32),
            jax.ShapeDtypeStruct((T, H), jnp.float32),
        ],
        scratch_types=[
            pltpu.VMEM((T // NW,), jnp.int32),
            pltpu.VMEM((T // NW,), jnp.int32),
            pltpu.VMEM((T // NW, H), jnp.float32),
            pltpu.VMEM((T // NW, H), jnp.float32),
            pltpu.SemaphoreType.DMA,
        ],
    )(y, pos0, pos1)


def _tc_gate_up(block_expert, x_sorted, W_gate_up, interpret=False):
    grid_spec = pltpu.PrefetchScalarGridSpec(
        num_scalar_prefetch=1,
        grid=(NI, NB),
        in_specs=[
            pl.BlockSpec((BM, H), lambda j, b, be: (b, 0)),
            pl.BlockSpec((1, H, BI), lambda j, b, be: (be[b], 0, j)),
            pl.BlockSpec((1, H, BI), lambda j, b, be: (be[b], 0, NI + j)),
        ],
        out_specs=pl.BlockSpec((BM, BI), lambda j, b, be: (b, j)),
    )
    return pl.pallas_call(
        _k1_body,
        grid_spec=grid_spec,
        out_shape=jax.ShapeDtypeStruct((P, I), jnp.bfloat16),
        compiler_params=pltpu.CompilerParams(
            dimension_semantics=("arbitrary", "arbitrary")),
        interpret=interpret,
    )(block_expert, x_sorted, W_gate_up, W_gate_up)


def _tc_down(block_expert, inter, W_down, interpret=False):
    grid_spec = pltpu.PrefetchScalarGridSpec(
        num_scalar_prefetch=1,
        grid=(NB,),
        in_specs=[
            pl.BlockSpec((BM, I), lambda b, be: (b, 0)),
            pl.BlockSpec((1, I, H), lambda b, be: (be[b], 0, 0)),
        ],
        out_specs=pl.BlockSpec((BM, H), lambda b, be: (b, 0)),
    )
    return pl.pallas_call(
        _k2_body,
        grid_spec=grid_spec,
        out_shape=jax.ShapeDtypeStruct((P, H), jnp.float32),
        compiler_params=pltpu.CompilerParams(
            dimension_semantics=("arbitrary",)),
        interpret=interpret,
    )(block_expert, inter, W_down)


def _tc_combine(y0, y1, expert_affinities, idx32, interpret=False):
    return pl.pallas_call(
        _k4_body,
        grid=(T // BT,),
        in_specs=[
            pl.BlockSpec((BT, H), lambda b: (b, 0)),
            pl.BlockSpec((BT, H), lambda b: (b, 0)),
            pl.BlockSpec((BT, E), lambda b: (b, 0)),
            pl.BlockSpec((BT, TOPK), lambda b: (b, 0)),
        ],
        out_specs=pl.BlockSpec((BT, H), lambda b: (b, 0)),
        out_shape=jax.ShapeDtypeStruct((T, H), jnp.float32),
        interpret=interpret,
    )(y0, y1, expert_affinities, idx32)


def _routing_metadata(idx32):
    flat_e = idx32.reshape(-1)
    n = flat_e.shape[0]
    perm = jnp.argsort(flat_e).astype(jnp.int32)
    sorted_e = flat_e[perm]
    counts = jnp.bincount(flat_e, length=E).astype(jnp.int32)
    padded = ((counts + BM - 1) // BM) * BM
    pend = jnp.cumsum(padded)
    pstart = pend - padded
    gend = jnp.cumsum(counts)
    gstart = gend - counts
    dest = (pstart[sorted_e] + jnp.arange(n, dtype=jnp.int32)
            - gstart[sorted_e]).astype(jnp.int32)
    pad_fill = jnp.arange(P, dtype=jnp.int32) % T
    tok_map = pad_fill.at[dest].set(perm // 2)
    pos_flat = jnp.zeros((n,), jnp.int32).at[perm].set(dest)
    pos0 = pos_flat[0::2]
    pos1 = pos_flat[1::2]
    block_starts = jnp.arange(NB, dtype=jnp.int32) * BM
    block_expert = jnp.minimum(
        jnp.searchsorted(pend, block_starts, side="right"), E - 1
    ).astype(jnp.int32)
    return tok_map, pos0, pos1, block_expert


def kernel(hidden_states, expert_affinities, expert_index, W_gate_up, W_down):
    idx32 = expert_index.astype(jnp.int32)
    tok_map, pos0, pos1, block_expert = _routing_metadata(idx32)
    x_sorted = _sc_gather_rows(hidden_states, tok_map)
    inter = _tc_gate_up(block_expert, x_sorted, W_gate_up)
    y = _tc_down(block_expert, inter, W_down)
    y0, y1 = _sc_gather_pair(y, pos0, pos1)
    return _tc_combine(y0, y1, expert_affinities, idx32)


# speedup vs baseline: 1.2879x; 1.1651x over previous
import jax
import jax.numpy as jnp
from jax import lax
from jax.experimental import pallas as pl
from jax.experimental.pallas import tpu as pltpu
from jax.experimental.pallas import tpu_sc as plsc

E = 8
TOPK = 2
H = 768
I = 3072
T = 2048

BM = 128
P = TOPK * T + E * BM
NB = P // BM
BI = 1024
NI = I // BI
BT = 256

NC = 2
NS = 16
NW = NC * NS
SC_CHUNK = 32


def _k0_body(hs_hbm, tok_hbm, xs_out, idx_v, rows_v, sem):
    wid = lax.axis_index("s") * NC + lax.axis_index("c")
    rows_per_w = P // NW
    base = wid * rows_per_w
    pltpu.sync_copy(tok_hbm.at[pl.ds(base, rows_per_w)], idx_v)
    c0 = pltpu.async_copy(hs_hbm.at[idx_v.at[pl.ds(0, 128)]],
                          rows_v.at[pl.ds(0, 128)], sem)
    c1 = pltpu.async_copy(hs_hbm.at[idx_v.at[pl.ds(128, 32)]],
                          rows_v.at[pl.ds(128, 32)], sem)
    c0.wait()
    c1.wait()
    pltpu.sync_copy(rows_v, xs_out.at[pl.ds(base, rows_per_w)])


def _k3_body(y_hbm, pos0_hbm, pos1_hbm, y0_out, y1_out,
             p0_v, p1_v, r0_v, r1_v, sem):
    wid = lax.axis_index("s") * NC + lax.axis_index("c")
    toks_per_w = T // NW
    base = wid * toks_per_w
    pltpu.sync_copy(pos0_hbm.at[pl.ds(base, toks_per_w)], p0_v)
    pltpu.sync_copy(pos1_hbm.at[pl.ds(base, toks_per_w)], p1_v)
    c0 = pltpu.async_copy(y_hbm.at[p0_v], r0_v, sem)
    c1 = pltpu.async_copy(y_hbm.at[p1_v], r1_v, sem)
    c0.wait()
    pltpu.sync_copy(r0_v, y0_out.at[pl.ds(base, toks_per_w)])
    c1.wait()
    pltpu.sync_copy(r1_v, y1_out.at[pl.ds(base, toks_per_w)])


def _k1_body(be_ref, x_ref, wg_ref, wu_ref, o_ref):
    x = x_ref[...].astype(jnp.bfloat16)
    g = jnp.dot(x, wg_ref[0].astype(jnp.bfloat16),
                preferred_element_type=jnp.float32)
    u = jnp.dot(x, wu_ref[0].astype(jnp.bfloat16),
                preferred_element_type=jnp.float32)
    o_ref[...] = (g * lax.logistic(g) * u).astype(jnp.bfloat16)


def _k2_body(be_ref, inter_ref, wd_ref, o_ref):
    o_ref[...] = jnp.dot(inter_ref[...], wd_ref[0].astype(jnp.bfloat16),
                         preferred_element_type=jnp.float32)


def _k4_body(y0_ref, y1_ref, aff_ref, idx_ref, o_ref):
    aff = aff_ref[...]
    i0 = idx_ref[:, 0:1]
    i1 = idx_ref[:, 1:2]
    a0 = jnp.zeros((BT, 1), jnp.float32)
    a1 = jnp.zeros((BT, 1), jnp.float32)
    for e in range(E):
        a0 = a0 + jnp.where(i0 == e, aff[:, e:e + 1], 0.0)
        a1 = a1 + jnp.where(i1 == e, aff[:, e:e + 1], 0.0)
    dup = i0 == i1
    denom = jnp.abs(a0) + jnp.where(dup, 0.0, jnp.abs(a1))
    denom = jnp.maximum(denom, 1e-12)
    w0 = a0 / denom
    w1 = jnp.where(dup, 0.0, a1 / denom)
    o_ref[...] = w0 * y0_ref[...] + w1 * y1_ref[...]


def _sc_gather_rows(hidden_states, tok_map):
    mesh = plsc.VectorSubcoreMesh(core_axis_name="c", subcore_axis_name="s")
    return pl.kernel(
        _k0_body,
        mesh=mesh,
        out_type=jax.ShapeDtypeStruct((P, H), jnp.float32),
        scratch_types=[
            pltpu.VMEM((P // NW,), jnp.int32),
            pltpu.VMEM((P // NW, H), jnp.float32),
            pltpu.SemaphoreType.DMA,
        ],
    )(hidden_states, tok_map)


def _sc_gather_pair(y, pos0, pos1):
    mesh = plsc.VectorSubcoreMesh(core_axis_name="c", subcore_axis_name="s")
    return pl.kernel(
        _k3_body,
        mesh=mesh,
        out_type=[
            jax.ShapeDtypeStruct((T, H), jnp.float32),
            jax.ShapeDtypeStruct((T, H), jnp.float32),
        ],
        scratch_types=[
            pltpu.VMEM((T // NW,), jnp.int32),
            pltpu.VMEM((T // NW,), jnp.int32),
            pltpu.VMEM((T // NW, H), jnp.float32),
            pltpu.VMEM((T // NW, H), jnp.float32),
            pltpu.SemaphoreType.DMA,
        ],
    )(y, pos0, pos1)


def _tc_gate_up(block_expert, x_sorted, W_gate_up, interpret=False):
    grid_spec = pltpu.PrefetchScalarGridSpec(
        num_scalar_prefetch=1,
        grid=(NI, NB),
        in_specs=[
            pl.BlockSpec((BM, H), lambda j, b, be: (b, 0)),
            pl.BlockSpec((1, H, BI), lambda j, b, be: (be[b], 0, j)),
            pl.BlockSpec((1, H, BI), lambda j, b, be: (be[b], 0, NI + j)),
        ],
        out_specs=pl.BlockSpec((BM, BI), lambda j, b, be: (b, j)),
    )
    return pl.pallas_call(
        _k1_body,
        grid_spec=grid_spec,
        out_shape=jax.ShapeDtypeStruct((P, I), jnp.bfloat16),
        compiler_params=pltpu.CompilerParams(
            dimension_semantics=("arbitrary", "arbitrary")),
        interpret=interpret,
    )(block_expert, x_sorted, W_gate_up, W_gate_up)


def _tc_down(block_expert, inter, W_down, interpret=False):
    grid_spec = pltpu.PrefetchScalarGridSpec(
        num_scalar_prefetch=1,
        grid=(NB,),
        in_specs=[
            pl.BlockSpec((BM, I), lambda b, be: (b, 0)),
            pl.BlockSpec((1, I, H), lambda b, be: (be[b], 0, 0)),
        ],
        out_specs=pl.BlockSpec((BM, H), lambda b, be: (b, 0)),
    )
    return pl.pallas_call(
        _k2_body,
        grid_spec=grid_spec,
        out_shape=jax.ShapeDtypeStruct((P, H), jnp.float32),
        compiler_params=pltpu.CompilerParams(
            dimension_semantics=("arbitrary",)),
        interpret=interpret,
    )(block_expert, inter, W_down)


def _tc_combine(y0, y1, expert_affinities, idx32, interpret=False):
    return pl.pallas_call(
        _k4_body,
        grid=(T // BT,),
        in_specs=[
            pl.BlockSpec((BT, H), lambda b: (b, 0)),
            pl.BlockSpec((BT, H), lambda b: (b, 0)),
            pl.BlockSpec((BT, E), lambda b: (b, 0)),
            pl.BlockSpec((BT, TOPK), lambda b: (b, 0)),
        ],
        out_specs=pl.BlockSpec((BT, H), lambda b: (b, 0)),
        out_shape=jax.ShapeDtypeStruct((T, H), jnp.float32),
        interpret=interpret,
    )(y0, y1, expert_affinities, idx32)


def _routing_metadata(idx32):
    flat_e = idx32.reshape(-1)
    n = flat_e.shape[0]
    eids = jnp.arange(E, dtype=jnp.int32)
    onehot = (flat_e[:, None] == eids[None, :]).astype(jnp.int32)
    csum = jnp.cumsum(onehot, axis=0)
    counts = csum[-1]
    padded = ((counts + BM - 1) // BM) * BM
    pend = jnp.cumsum(padded)
    pstart = pend - padded
    rank = jnp.sum(jnp.where(onehot == 1, csum - 1, 0), axis=1)
    seg = jnp.sum(jnp.where(onehot == 1, pstart[None, :], 0), axis=1)
    dest = (seg + rank).astype(jnp.int32)
    pad_fill = jnp.arange(P, dtype=jnp.int32) % T
    tok_map = pad_fill.at[dest].set(
        jnp.arange(n, dtype=jnp.int32) // 2)
    pos0 = dest[0::2]
    pos1 = dest[1::2]
    block_starts = jnp.arange(NB, dtype=jnp.int32) * BM
    block_expert = jnp.sum(
        (pend[None, :] <= block_starts[:, None]).astype(jnp.int32), axis=1)
    block_expert = jnp.minimum(block_expert, E - 1)
    return tok_map, pos0, pos1, block_expert


def kernel(hidden_states, expert_affinities, expert_index, W_gate_up, W_down):
    idx32 = expert_index.astype(jnp.int32)
    tok_map, pos0, pos1, block_expert = _routing_metadata(idx32)
    x_sorted = _sc_gather_rows(hidden_states, tok_map)
    inter = _tc_gate_up(block_expert, x_sorted, W_gate_up)
    y = _tc_down(block_expert, inter, W_down)
    y0, y1 = _sc_gather_pair(y, pos0, pos1)
    return _tc_combine(y0, y1, expert_affinities, idx32)


# speedup vs baseline: 1.3779x; 1.0699x over previous
import jax
import jax.numpy as jnp
from jax import lax
from jax.experimental import pallas as pl
from jax.experimental.pallas import tpu as pltpu
from jax.experimental.pallas import tpu_sc as plsc

E = 8
TOPK = 2
H = 768
I = 3072
T = 2048

BM = 128
P = TOPK * T + E * BM
NB = P // BM
BI = 1024
NI = I // BI
BT = 256

NC = 2
NS = 16
NW = NC * NS
SC_CHUNK = 32


def _k0_body(hs_hbm, pos0_hbm, pos1_hbm, xs_out, p0_v, p1_v, rows_v, sem):
    wid = lax.axis_index("s") * NC + lax.axis_index("c")
    tpw = T // NW
    base = wid * tpw
    pltpu.sync_copy(pos0_hbm.at[pl.ds(base, tpw)], p0_v)
    pltpu.sync_copy(pos1_hbm.at[pl.ds(base, tpw)], p1_v)
    pltpu.sync_copy(hs_hbm.at[pl.ds(base, tpw)], rows_v)
    c0 = pltpu.async_copy(rows_v, xs_out.at[p0_v], sem)
    c1 = pltpu.async_copy(rows_v, xs_out.at[p1_v], sem)
    c0.wait()
    c1.wait()


def _k3_body(y_hbm, pos0_hbm, pos1_hbm, y0_out, y1_out,
             p0_v, p1_v, r0_v, r1_v, sem):
    wid = lax.axis_index("s") * NC + lax.axis_index("c")
    toks_per_w = T // NW
    base = wid * toks_per_w
    pltpu.sync_copy(pos0_hbm.at[pl.ds(base, toks_per_w)], p0_v)
    pltpu.sync_copy(pos1_hbm.at[pl.ds(base, toks_per_w)], p1_v)
    c0 = pltpu.async_copy(y_hbm.at[p0_v], r0_v, sem)
    c1 = pltpu.async_copy(y_hbm.at[p1_v], r1_v, sem)
    c0.wait()
    pltpu.sync_copy(r0_v, y0_out.at[pl.ds(base, toks_per_w)])
    c1.wait()
    pltpu.sync_copy(r1_v, y1_out.at[pl.ds(base, toks_per_w)])


def _k1_body(be_ref, x_ref, wg_ref, wu_ref, o_ref):
    x = x_ref[...].astype(jnp.bfloat16)
    g = jnp.dot(x, wg_ref[0].astype(jnp.bfloat16),
                preferred_element_type=jnp.float32)
    u = jnp.dot(x, wu_ref[0].astype(jnp.bfloat16),
                preferred_element_type=jnp.float32)
    o_ref[...] = (g * lax.logistic(g) * u).astype(jnp.bfloat16)


def _k2_body(be_ref, inter_ref, wd_ref, o_ref):
    o_ref[...] = jnp.dot(inter_ref[...], wd_ref[0].astype(jnp.bfloat16),
                         preferred_element_type=jnp.float32)


def _k4_body(y0_ref, y1_ref, aff_ref, idx_ref, o_ref):
    aff = aff_ref[...]
    i0 = idx_ref[:, 0:1]
    i1 = idx_ref[:, 1:2]
    a0 = jnp.zeros((BT, 1), jnp.float32)
    a1 = jnp.zeros((BT, 1), jnp.float32)
    for e in range(E):
        a0 = a0 + jnp.where(i0 == e, aff[:, e:e + 1], 0.0)
        a1 = a1 + jnp.where(i1 == e, aff[:, e:e + 1], 0.0)
    dup = i0 == i1
    denom = jnp.abs(a0) + jnp.where(dup, 0.0, jnp.abs(a1))
    denom = jnp.maximum(denom, 1e-12)
    w0 = a0 / denom
    w1 = jnp.where(dup, 0.0, a1 / denom)
    o_ref[...] = w0 * y0_ref[...] + w1 * y1_ref[...]


def _sc_scatter_rows(hidden_states, pos0, pos1):
    mesh = plsc.VectorSubcoreMesh(core_axis_name="c", subcore_axis_name="s")
    return pl.kernel(
        _k0_body,
        mesh=mesh,
        out_type=jax.ShapeDtypeStruct((P, H), jnp.float32),
        scratch_types=[
            pltpu.VMEM((T // NW,), jnp.int32),
            pltpu.VMEM((T // NW,), jnp.int32),
            pltpu.VMEM((T // NW, H), jnp.float32),
            pltpu.SemaphoreType.DMA,
        ],
    )(hidden_states, pos0, pos1)


def _sc_gather_pair(y, pos0, pos1):
    mesh = plsc.VectorSubcoreMesh(core_axis_name="c", subcore_axis_name="s")
    return pl.kernel(
        _k3_body,
        mesh=mesh,
        out_type=[
            jax.ShapeDtypeStruct((T, H), jnp.float32),
            jax.ShapeDtypeStruct((T, H), jnp.float32),
        ],
        scratch_types=[
            pltpu.VMEM((T // NW,), jnp.int32),
            pltpu.VMEM((T // NW,), jnp.int32),
            pltpu.VMEM((T // NW, H), jnp.float32),
            pltpu.VMEM((T // NW, H), jnp.float32),
            pltpu.SemaphoreType.DMA,
        ],
    )(y, pos0, pos1)


def _tc_gate_up(block_expert, x_sorted, W_gate_up, interpret=False):
    grid_spec = pltpu.PrefetchScalarGridSpec(
        num_scalar_prefetch=1,
        grid=(NI, NB),
        in_specs=[
            pl.BlockSpec((BM, H), lambda j, b, be: (b, 0)),
            pl.BlockSpec((1, H, BI), lambda j, b, be: (be[b], 0, j)),
            pl.BlockSpec((1, H, BI), lambda j, b, be: (be[b], 0, NI + j)),
        ],
        out_specs=pl.BlockSpec((BM, BI), lambda j, b, be: (b, j)),
    )
    return pl.pallas_call(
        _k1_body,
        grid_spec=grid_spec,
        out_shape=jax.ShapeDtypeStruct((P, I), jnp.bfloat16),
        compiler_params=pltpu.CompilerParams(
            dimension_semantics=("arbitrary", "arbitrary")),
        interpret=interpret,
    )(block_expert, x_sorted, W_gate_up, W_gate_up)


def _tc_down(block_expert, inter, W_down, interpret=False):
    grid_spec = pltpu.PrefetchScalarGridSpec(
        num_scalar_prefetch=1,
        grid=(NB,),
        in_specs=[
            pl.BlockSpec((BM, I), lambda b, be: (b, 0)),
            pl.BlockSpec((1, I, H), lambda b, be: (be[b], 0, 0)),
        ],
        out_specs=pl.BlockSpec((BM, H), lambda b, be: (b, 0)),
    )
    return pl.pallas_call(
        _k2_body,
        grid_spec=grid_spec,
        out_shape=jax.ShapeDtypeStruct((P, H), jnp.float32),
        compiler_params=pltpu.CompilerParams(
            dimension_semantics=("arbitrary",)),
        interpret=interpret,
    )(block_expert, inter, W_down)


def _tc_combine(y0, y1, expert_affinities, idx32, interpret=False):
    return pl.pallas_call(
        _k4_body,
        grid=(T // BT,),
        in_specs=[
            pl.BlockSpec((BT, H), lambda b: (b, 0)),
            pl.BlockSpec((BT, H), lambda b: (b, 0)),
            pl.BlockSpec((BT, E), lambda b: (b, 0)),
            pl.BlockSpec((BT, TOPK), lambda b: (b, 0)),
        ],
        out_specs=pl.BlockSpec((BT, H), lambda b: (b, 0)),
        out_shape=jax.ShapeDtypeStruct((T, H), jnp.float32),
        interpret=interpret,
    )(y0, y1, expert_affinities, idx32)


def _routing_metadata(idx32):
    flat_e = idx32.reshape(-1)
    n = flat_e.shape[0]
    eids = jnp.arange(E, dtype=jnp.int32)
    onehot = (flat_e[:, None] == eids[None, :]).astype(jnp.int32)
    csum = jnp.cumsum(onehot, axis=0)
    counts = csum[-1]
    padded = ((counts + BM - 1) // BM) * BM
    pend = jnp.cumsum(padded)
    pstart = pend - padded
    rank = jnp.sum(jnp.where(onehot == 1, csum - 1, 0), axis=1)
    seg = jnp.sum(jnp.where(onehot == 1, pstart[None, :], 0), axis=1)
    dest = (seg + rank).astype(jnp.int32)
    pos0 = dest[0::2]
    pos1 = dest[1::2]
    block_starts = jnp.arange(NB, dtype=jnp.int32) * BM
    block_expert = jnp.sum(
        (pend[None, :] <= block_starts[:, None]).astype(jnp.int32), axis=1)
    block_expert = jnp.minimum(block_expert, E - 1)
    return pos0, pos1, block_expert


def kernel(hidden_states, expert_affinities, expert_index, W_gate_up, W_down):
    idx32 = expert_index.astype(jnp.int32)
    pos0, pos1, block_expert = _routing_metadata(idx32)
    x_sorted = _sc_scatter_rows(hidden_states, pos0, pos1)
    inter = _tc_gate_up(block_expert, x_sorted, W_gate_up)
    y = _tc_down(block_expert, inter, W_down)
    y0, y1 = _sc_gather_pair(y, pos0, pos1)
    return _tc_combine(y0, y1, expert_affinities, idx32)
